# baseline (speedup 1.0000x reference)
import functools

import jax
import jax.numpy as jnp
from jax import lax
from jax.experimental import pallas as pl
from jax.experimental.pallas import tpu as pltpu
from jax.experimental.pallas import tpu_sc as plsc

N = 10000
E = 320000
D = 128
G = 64

NC = 2
NS = 16
NW = NC * NS

CH = 128
N_PAD = 10240
ROWS_PER_TILE = N_PAD // NS
MAIN_CHUNKS = E // CH
MAIN_PER_W = 72
XTRA_PER_W = 8
XTRA_CHUNKS = NW * XTRA_PER_W
PHASES = 2
PH_CHUNKS = 40

@functools.lru_cache(maxsize=None)
def _get_sc_aggregate():
    mesh = plsc.VectorSubcoreMesh(
        core_axis_name="c", subcore_axis_name="s", num_cores=NC, num_subcores=NS
    )

    @functools.partial(
        pl.kernel,
        out_type=jax.ShapeDtypeStruct((NC, N_PAD, D), jnp.float32),
        mesh=mesh,
        scratch_types=[
            pltpu.VMEM((2, PH_CHUNKS, CH), jnp.int32),
            pltpu.VMEM((CH, D), jnp.float32),
            pltpu.VMEM((CH, D), jnp.float32),
            pltpu.VMEM_SHARED((N_PAD, D), jnp.float32),
            pltpu.SemaphoreType.DMA,
            pltpu.SemaphoreType.DMA,
            pltpu.SemaphoreType.DMA,
        ],
    )
    def _sc_aggregate(edge_hbm, xtra_hbm, node_hbm, out_hbm,
                      idx_v, rows0_v, rows1_v, agg_sh,
                      sem_i, sem_g0, sem_g1):
        c = lax.axis_index("c")
        s = lax.axis_index("s")
        wid = s * NC + c

        def load_idx(phase):
            base = wid * MAIN_PER_W
            if phase == 0:
                return (pltpu.async_copy(
                    edge_hbm.at[:, pl.ds(base, PH_CHUNKS), :],
                    idx_v.at[:, pl.ds(0, PH_CHUNKS), :], sem_i),)
            n_main = MAIN_PER_W - PH_CHUNKS
            cp0 = pltpu.async_copy(
                edge_hbm.at[:, pl.ds(base + PH_CHUNKS, n_main), :],
                idx_v.at[:, pl.ds(0, n_main), :], sem_i)
            cp1 = pltpu.async_copy(
                xtra_hbm.at[:, pl.ds(XTRA_PER_W * wid, XTRA_PER_W), :],
                idx_v.at[:, pl.ds(n_main, XTRA_PER_W), :], sem_i)
            return (cp0, cp1)

        idx_cp = load_idx(0)

        z = jnp.zeros((16,), jnp.float32)

        def zero_row(i, carry):
            for j in range(D // 16):
                rows0_v[i, pl.ds(j * 16, 16)] = z
            return carry

        lax.fori_loop(0, CH, zero_row, 0)
        for m in range(ROWS_PER_TILE // CH):
            pltpu.sync_copy(
                rows0_v, agg_sh.at[pl.ds(s * ROWS_PER_TILE + m * CH, CH), :]
            )
        plsc.subcore_barrier()

        bufs = ((rows0_v, sem_g0), (rows1_v, sem_g1))

        def issue_gather(j, buf, sem):
            pltpu.async_copy(node_hbm.at[idx_v.at[0, j]], buf, sem)

        for phase in range(PHASES):
            for cp in idx_cp:
                cp.wait()
            for b, (buf, sem) in enumerate(bufs):
                issue_gather(b, buf, sem)

            def group_body(g, carry):
                for b, (buf, sem) in enumerate(bufs):
                    j = g * 2 + b
                    pltpu.make_async_copy(node_hbm.at[idx_v.at[0, j]], buf, sem).wait()
                    pltpu.sync_copy(buf, agg_sh.at[idx_v.at[1, j]], add=True)
                    nxt = j + 2

                    @pl.when(nxt < PH_CHUNKS)
                    def _prefetch():
                        issue_gather(nxt, buf, sem)

                return carry

            lax.fori_loop(0, PH_CHUNKS // 2, group_body, 0)
            if phase + 1 < PHASES:
                idx_cp = load_idx(phase + 1)
        plsc.subcore_barrier()

        pltpu.sync_copy(
            agg_sh.at[pl.ds(s * ROWS_PER_TILE, ROWS_PER_TILE), :],
            out_hbm.at[c, pl.ds(s * ROWS_PER_TILE, ROWS_PER_TILE), :],
        )

    return _sc_aggregate


R = 2000
NBLK = N // R
SB = 16
SBR = R // SB


_dot = functools.partial(
    jnp.dot,
    preferred_element_type=jnp.float32,
    precision=lax.Precision.HIGHEST,
)


def _tc_self_body(node_ref, ws_ref, xs_ref):
    xs_ref[...] = _dot(node_ref[...], ws_ref[...])


_tc_self = pl.pallas_call(
    _tc_self_body,
    grid=(NBLK,),
    in_specs=[
        pl.BlockSpec((R, D), lambda i: (i, 0)),
        pl.BlockSpec((D, D), lambda i: (0, 0)),
    ],
    out_specs=pl.BlockSpec((R, D), lambda i: (i, 0)),
    out_shape=jax.ShapeDtypeStruct((N, D), jnp.float32),
)


def _tc_body(batch_ref, xs_ref, agg_ref, wn_ref,
             w1_ref, b1_ref, w2_ref, b2_ref, out_ref):
    i = pl.program_id(0)

    a = agg_ref[0]
    for p in range(1, NC):
        a = a + agg_ref[p]
    h = jnp.maximum(xs_ref[...] + _dot(a, wn_ref[...]), 0.0)
    h = jnp.maximum(_dot(h, w1_ref[...]) + b1_ref[...], 0.0)
    h = _dot(h, w2_ref[...]) + b2_ref[...]

    @pl.when(i == 0)
    def _init():
        out_ref[...] = jnp.full((G, D), -jnp.inf, jnp.float32)

    b = batch_ref[...]
    for k in range(SB):
        bs = b[k * SBR:(k + 1) * SBR]
        hs = h[k * SBR:(k + 1) * SBR]
        g_lo = jnp.min(bs)
        g_hi = jnp.max(bs)

        def seg_body(g, carry, bs=bs, hs=hs):
            v = jnp.where(bs == g, hs, -jnp.inf)
            m = jnp.max(v, axis=0, keepdims=True)
            out_ref[pl.ds(g, 1), :] = jnp.maximum(out_ref[pl.ds(g, 1), :], m)
            return carry

        lax.fori_loop(g_lo, g_hi + 1, seg_body, 0)


_tc_forward = pl.pallas_call(
    _tc_body,
    grid=(NBLK,),
    in_specs=[
        pl.BlockSpec((R, 1), lambda i: (i, 0)),
        pl.BlockSpec((R, D), lambda i: (i, 0)),
        pl.BlockSpec((NC, R, D), lambda i: (0, i, 0)),
        pl.BlockSpec((D, D), lambda i: (0, 0)),
        pl.BlockSpec((D, D), lambda i: (0, 0)),
        pl.BlockSpec((1, D), lambda i: (0, 0)),
        pl.BlockSpec((D, D), lambda i: (0, 0)),
        pl.BlockSpec((1, D), lambda i: (0, 0)),
    ],
    out_specs=pl.BlockSpec((G, D), lambda i: (0, 0)),
    out_shape=jax.ShapeDtypeStruct((G, D), jnp.float32),
)


@jax.jit
def kernel(node, edge_index, batch, W_self, W_neigh, W1, b1, W2, b2):
    ei3 = edge_index.astype(jnp.int32).reshape(2, MAIN_CHUNKS, CH)
    nleft = MAIN_CHUNKS - NW * MAIN_PER_W
    npad = XTRA_CHUNKS - nleft
    pad_i = jnp.arange(npad * CH, dtype=jnp.int32)
    pad2 = jnp.stack([
        (pad_i % N).reshape(npad, CH),
        N + (pad_i % (N_PAD - N)).reshape(npad, CH),
    ])
    xtra = jnp.concatenate([ei3[:, NW * MAIN_PER_W:, :], pad2], axis=1)

    agg = _get_sc_aggregate()(ei3, xtra, node)
    xs = _tc_self(node, W_self)

    return _tc_forward(
        batch.astype(jnp.int32).reshape(N, 1),
        xs,
        agg,
        W_neigh, W1,
        b1.reshape(1, D), W2, b2.reshape(1, D),
    )


# speedup vs baseline: 1.0608x; 1.0608x over previous
import functools

import jax
import jax.numpy as jnp
from jax import lax
from jax.experimental import pallas as pl
from jax.experimental.pallas import tpu as pltpu
from jax.experimental.pallas import tpu_sc as plsc

N = 10000
E = 320000
D = 128
G = 64

NC = 2
NS = 16
NW = NC * NS

CH = 128
N_PAD = 10240
ROWS_PER_TILE = N_PAD // NS
MAIN_CHUNKS = E // CH
MAIN_PER_W = 72
XTRA_PER_W = 8
XTRA_CHUNKS = NW * XTRA_PER_W
PHASES = 2
PH_CHUNKS = 40

@functools.lru_cache(maxsize=None)
def _get_sc_aggregate():
    mesh = plsc.VectorSubcoreMesh(
        core_axis_name="c", subcore_axis_name="s", num_cores=NC, num_subcores=NS
    )

    @functools.partial(
        pl.kernel,
        out_type=jax.ShapeDtypeStruct((NC, N_PAD, D), jnp.float32),
        mesh=mesh,
        scratch_types=[
            pltpu.VMEM((2, PH_CHUNKS, CH), jnp.int32),
            pltpu.VMEM((CH, D), jnp.float32),
            pltpu.VMEM((CH, D), jnp.float32),
            pltpu.VMEM_SHARED((N_PAD, D), jnp.float32),
            pltpu.SemaphoreType.DMA,
            pltpu.SemaphoreType.DMA,
            pltpu.SemaphoreType.DMA,
        ],
    )
    def _sc_aggregate(edge_hbm, xtra_hbm, node_hbm, out_hbm,
                      idx_v, rows0_v, rows1_v, agg_sh,
                      sem_i, sem_g0, sem_g1):
        c = lax.axis_index("c")
        s = lax.axis_index("s")
        wid = s * NC + c

        def load_idx(phase):
            base = wid * MAIN_PER_W
            if phase == 0:
                return (pltpu.async_copy(
                    edge_hbm.at[:, pl.ds(base, PH_CHUNKS), :],
                    idx_v.at[:, pl.ds(0, PH_CHUNKS), :], sem_i),)
            n_main = MAIN_PER_W - PH_CHUNKS
            cp0 = pltpu.async_copy(
                edge_hbm.at[:, pl.ds(base + PH_CHUNKS, n_main), :],
                idx_v.at[:, pl.ds(0, n_main), :], sem_i)
            cp1 = pltpu.async_copy(
                xtra_hbm.at[:, pl.ds(XTRA_PER_W * wid, XTRA_PER_W), :],
                idx_v.at[:, pl.ds(n_main, XTRA_PER_W), :], sem_i)
            return (cp0, cp1)

        idx_cp = load_idx(0)

        z = jnp.zeros((16,), jnp.float32)

        def zero_row(i, carry):
            for j in range(D // 16):
                rows0_v[i, pl.ds(j * 16, 16)] = z
            return carry

        lax.fori_loop(0, CH, zero_row, 0)
        for m in range(ROWS_PER_TILE // CH):
            pltpu.sync_copy(
                rows0_v, agg_sh.at[pl.ds(s * ROWS_PER_TILE + m * CH, CH), :]
            )
        plsc.subcore_barrier()

        bufs = ((rows0_v, sem_g0), (rows1_v, sem_g1))

        def issue_gather(j, buf, sem):
            pltpu.async_copy(node_hbm.at[idx_v.at[0, j]], buf, sem)

        for phase in range(PHASES):
            for cp in idx_cp:
                cp.wait()
            for b, (buf, sem) in enumerate(bufs):
                issue_gather(b, buf, sem)

            def group_body(g, carry):
                for b, (buf, sem) in enumerate(bufs):
                    j = g * 2 + b
                    pltpu.make_async_copy(node_hbm.at[idx_v.at[0, j]], buf, sem).wait()
                    pltpu.sync_copy(buf, agg_sh.at[idx_v.at[1, j]], add=True)
                    nxt = j + 2

                    @pl.when(nxt < PH_CHUNKS)
                    def _prefetch():
                        issue_gather(nxt, buf, sem)

                return carry

            lax.fori_loop(0, PH_CHUNKS // 2, group_body, 0)
            if phase + 1 < PHASES:
                idx_cp = load_idx(phase + 1)
        plsc.subcore_barrier()

        pltpu.sync_copy(
            agg_sh.at[pl.ds(s * ROWS_PER_TILE, ROWS_PER_TILE), :],
            out_hbm.at[c, pl.ds(s * ROWS_PER_TILE, ROWS_PER_TILE), :],
        )

    return _sc_aggregate


R = 2000
NBLK = N // R
SB = 8
SBR = R // SB


_dot = functools.partial(
    jnp.dot,
    preferred_element_type=jnp.float32,
    precision=lax.Precision.HIGHEST,
)


def _tc_self_body(node_ref, ws_ref, xs_ref):
    xs_ref[...] = _dot(node_ref[...], ws_ref[...])


_tc_self = pl.pallas_call(
    _tc_self_body,
    grid=(NBLK,),
    in_specs=[
        pl.BlockSpec((R, D), lambda i: (i, 0)),
        pl.BlockSpec((D, D), lambda i: (0, 0)),
    ],
    out_specs=pl.BlockSpec((R, D), lambda i: (i, 0)),
    out_shape=jax.ShapeDtypeStruct((N, D), jnp.float32),
)


def _tc_body(batch_ref, xs_ref, agg_ref, wn_ref,
             w1_ref, b1_ref, w2_ref, b2_ref, out_ref):
    i = pl.program_id(0)

    a = agg_ref[0]
    for p in range(1, NC):
        a = a + agg_ref[p]
    h = jnp.maximum(xs_ref[...] + _dot(a, wn_ref[...]), 0.0)
    h = jnp.maximum(_dot(h, w1_ref[...]) + b1_ref[...], 0.0)
    h = _dot(h, w2_ref[...]) + b2_ref[...]

    b = batch_ref[...]
    giota = lax.broadcasted_iota(jnp.int32, (G, 1), 0)
    acc = jnp.full((G, D), -jnp.inf, jnp.float32)
    for k in range(SB):
        bs = b[k * SBR:(k + 1) * SBR]
        hs = h[k * SBR:(k + 1) * SBR]
        g_lo = jnp.min(bs)
        g_hi = jnp.max(bs)

        def seg_body(g, acc, bs=bs, hs=hs):
            v = jnp.where(bs == g, hs, -jnp.inf)
            m = jnp.max(v, axis=0, keepdims=True)
            return jnp.where(giota == g, jnp.maximum(acc, m), acc)

        acc = lax.fori_loop(g_lo, g_hi + 1, seg_body, acc)

    @pl.when(i == 0)
    def _init():
        out_ref[...] = jnp.full((G, D), -jnp.inf, jnp.float32)

    out_ref[...] = jnp.maximum(out_ref[...], acc)


_tc_forward = pl.pallas_call(
    _tc_body,
    grid=(NBLK,),
    in_specs=[
        pl.BlockSpec((R, 1), lambda i: (i, 0)),
        pl.BlockSpec((R, D), lambda i: (i, 0)),
        pl.BlockSpec((NC, R, D), lambda i: (0, i, 0)),
        pl.BlockSpec((D, D), lambda i: (0, 0)),
        pl.BlockSpec((D, D), lambda i: (0, 0)),
        pl.BlockSpec((1, D), lambda i: (0, 0)),
        pl.BlockSpec((D, D), lambda i: (0, 0)),
        pl.BlockSpec((1, D), lambda i: (0, 0)),
    ],
    out_specs=pl.BlockSpec((G, D), lambda i: (0, 0)),
    out_shape=jax.ShapeDtypeStruct((G, D), jnp.float32),
)


@jax.jit
def kernel(node, edge_index, batch, W_self, W_neigh, W1, b1, W2, b2):
    ei3 = edge_index.astype(jnp.int32).reshape(2, MAIN_CHUNKS, CH)
    nleft = MAIN_CHUNKS - NW * MAIN_PER_W
    npad = XTRA_CHUNKS - nleft
    pad_i = jnp.arange(npad * CH, dtype=jnp.int32)
    pad2 = jnp.stack([
        (pad_i % N).reshape(npad, CH),
        N + (pad_i % (N_PAD - N)).reshape(npad, CH),
    ])
    xtra = jnp.concatenate([ei3[:, NW * MAIN_PER_W:, :], pad2], axis=1)

    agg = _get_sc_aggregate()(ei3, xtra, node)
    xs = _tc_self(node, W_self)

    return _tc_forward(
        batch.astype(jnp.int32).reshape(N, 1),
        xs,
        agg,
        W_neigh, W1,
        b1.reshape(1, D), W2, b2.reshape(1, D),
    )
